# initial kernel scaffold (unmeasured)
import jax
import jax.numpy as jnp
from jax import lax
from jax.experimental import pallas as pl
from jax.experimental.pallas import tpu as pltpu

N_DEV = 16
B_LOC = 2
SQ = 256
SKV = 256
HQ = 64
DH = 64
D_MODEL = 512
H_PER = HQ // N_DEV
CHUNK = H_PER * DH


def kernel(x, Wq, K_ext, V_ext, Wo):
    my = lax.axis_index("i")
    b0 = my * B_LOC
    K_l = jnp.transpose(lax.dynamic_slice_in_dim(K_ext, b0, B_LOC, 0), (0, 2, 1, 3))
    V_l = jnp.transpose(lax.dynamic_slice_in_dim(V_ext, b0, B_LOC, 0), (0, 2, 1, 3))

    def body(x_ref, wq_ref, k_ref, v_ref, wo_ref, out_ref,
             wq_g, wo_g, wq_send, wq_recv, wo_send, wo_recv):
        my_i = lax.axis_index("i")
        left = lax.rem(my_i + N_DEV - 1, N_DEV)
        right = lax.rem(my_i + 1, N_DEV)

        out_ref[...] = jnp.zeros((B_LOC, SQ, D_MODEL), jnp.float32)

        qi = lax.broadcasted_iota(jnp.int32, (SQ, SKV), 0)
        ki = lax.broadcasted_iota(jnp.int32, (SQ, SKV), 1)
        mask = (jnp.abs(qi - ki) <= 128) | (ki < 32) | (qi < 32)

        barrier = pltpu.get_barrier_semaphore()
        for nbr in (left, right):
            pl.semaphore_signal(barrier, inc=1, device_id=(nbr,),
                                device_id_type=pl.DeviceIdType.MESH)
        pl.semaphore_wait(barrier, 2)

        wq_g[my_i] = wq_ref[...]
        wo_g[my_i] = wo_ref[...]

        def make_pair(j, h):
            r_wq = pltpu.make_async_remote_copy(
                src_ref=wq_g.at[j], dst_ref=wq_g.at[j],
                send_sem=wq_send.at[h], recv_sem=wq_recv.at[h],
                device_id=(right,), device_id_type=pl.DeviceIdType.MESH)
            r_wo = pltpu.make_async_remote_copy(
                src_ref=wo_g.at[j], dst_ref=wo_g.at[j],
                send_sem=wo_send.at[h], recv_sem=wo_recv.at[h],
                device_id=(right,), device_id_type=pl.DeviceIdType.MESH)
            return r_wq, r_wo

        def compute_chunk(j):
            wq_c = wq_g[j]
            wo_c = wo_g[j]
            for b in range(B_LOC):
                xb = x_ref[b]
                qc = jax.lax.dot_general(
                    xb, wq_c, (((1,), (0,)), ((), ())),
                    preferred_element_type=jnp.float32)
                ctx_parts = []
                for hh in range(H_PER):
                    hg = j * H_PER + hh
                    kh = k_ref[b, hg]
                    vh = v_ref[b, hg]
                    qh = qc[:, hh * DH:(hh + 1) * DH]
                    s = jax.lax.dot_general(
                        qh, kh, (((1,), (1,)), ((), ())),
                        preferred_element_type=jnp.float32) * 0.125
                    s = jnp.where(mask, s, -1e9)
                    m = jnp.max(s, axis=1, keepdims=True)
                    w = jnp.exp(s - m)
                    w = w / jnp.sum(w, axis=1, keepdims=True)
                    ctx_parts.append(jax.lax.dot_general(
                        w, vh, (((1,), (0,)), ((), ())),
                        preferred_element_type=jnp.float32))
                ctx = jnp.concatenate(ctx_parts, axis=1)
                contrib = jax.lax.dot_general(
                    ctx, wo_c, (((1,), (0,)), ((), ())),
                    preferred_element_type=jnp.float32)
                out_ref[b] = out_ref[b] + contrib

        for h in range(N_DEV):
            j = lax.rem(my_i - h + N_DEV, N_DEV)
            if h > 0:
                r_wq, r_wo = make_pair(j, h - 1)
                r_wq.wait_recv()
                r_wo.wait_recv()
            if h < N_DEV - 1:
                s_wq, s_wo = make_pair(j, h)
                s_wq.start()
                s_wo.start()
            compute_chunk(j)
            if h < N_DEV - 1:
                s_wq.wait_send()
                s_wo.wait_send()

    return pl.pallas_call(
        body,
        out_shape=jax.ShapeDtypeStruct((B_LOC, SQ, D_MODEL), jnp.float32),
        in_specs=[pl.BlockSpec(memory_space=pltpu.VMEM)] * 5,
        out_specs=pl.BlockSpec(memory_space=pltpu.VMEM),
        scratch_shapes=[
            pltpu.VMEM((N_DEV, D_MODEL, CHUNK), jnp.float32),
            pltpu.VMEM((N_DEV, CHUNK, D_MODEL), jnp.float32),
            pltpu.SemaphoreType.DMA((N_DEV - 1,)),
            pltpu.SemaphoreType.DMA((N_DEV - 1,)),
            pltpu.SemaphoreType.DMA((N_DEV - 1,)),
            pltpu.SemaphoreType.DMA((N_DEV - 1,)),
        ],
        compiler_params=pltpu.CompilerParams(collective_id=0),
    )(x, Wq, K_l, V_l, Wo)


# baseline (device time: 241390 ns/iter reference)
import jax
import jax.numpy as jnp
from jax import lax
from jax.experimental import pallas as pl
from jax.experimental.pallas import tpu as pltpu

N_DEV = 16
B_LOC = 2
SQ = 256
SKV = 256
HQ = 64
DH = 64
D_MODEL = 512
H_PER = HQ // N_DEV
CHUNK = H_PER * DH


def kernel(x, Wq, K_ext, V_ext, Wo):
    my = lax.axis_index("i")
    b0 = my * B_LOC
    K_l = jnp.transpose(lax.dynamic_slice_in_dim(K_ext, b0, B_LOC, 0), (0, 2, 1, 3))
    V_l = jnp.transpose(lax.dynamic_slice_in_dim(V_ext, b0, B_LOC, 0), (0, 2, 1, 3))

    def body(x_ref, wq_ref, k_ref, v_ref, wo_ref, out_ref,
             wq_g, wo_g, wq_send, wq_recv, wo_send, wo_recv):
        my_i = lax.axis_index("i")
        left = lax.rem(my_i + N_DEV - 1, N_DEV)
        right = lax.rem(my_i + 1, N_DEV)

        out_ref[...] = jnp.zeros((B_LOC, SQ, D_MODEL), jnp.float32)

        qi = lax.broadcasted_iota(jnp.int32, (SQ, SKV), 0)
        ki = lax.broadcasted_iota(jnp.int32, (SQ, SKV), 1)
        mask = (jnp.abs(qi - ki) <= 128) | (ki < 32) | (qi < 32)

        barrier = pltpu.get_barrier_semaphore()
        for nbr in (left, right):
            pl.semaphore_signal(barrier, inc=1, device_id=(nbr,),
                                device_id_type=pl.DeviceIdType.MESH)
        pl.semaphore_wait(barrier, 2)

        wq_g[my_i] = wq_ref[...]
        wo_g[my_i] = wo_ref[...]

        def make_pair(j, h):
            r_wq = pltpu.make_async_remote_copy(
                src_ref=wq_g.at[j], dst_ref=wq_g.at[j],
                send_sem=wq_send.at[h], recv_sem=wq_recv.at[h],
                device_id=(right,), device_id_type=pl.DeviceIdType.MESH)
            r_wo = pltpu.make_async_remote_copy(
                src_ref=wo_g.at[j], dst_ref=wo_g.at[j],
                send_sem=wo_send.at[h], recv_sem=wo_recv.at[h],
                device_id=(right,), device_id_type=pl.DeviceIdType.MESH)
            return r_wq, r_wo

        def compute_chunk(j):
            wq_c = wq_g[j]
            wo_c = wo_g[j]
            for b in range(B_LOC):
                xb = x_ref[b]
                qc = jax.lax.dot_general(
                    xb, wq_c, (((1,), (0,)), ((), ())),
                    preferred_element_type=jnp.float32)
                ctx_parts = []
                for hh in range(H_PER):
                    hg = j * H_PER + hh
                    kh = k_ref[b, hg]
                    vh = v_ref[b, hg]
                    qh = qc[:, hh * DH:(hh + 1) * DH]
                    s = jax.lax.dot_general(
                        qh, kh, (((1,), (1,)), ((), ())),
                        preferred_element_type=jnp.float32) * 0.125
                    s = jnp.where(mask, s, -1e9)
                    m = jnp.max(s, axis=1, keepdims=True)
                    w = jnp.exp(s - m)
                    w = w / jnp.sum(w, axis=1, keepdims=True)
                    ctx_parts.append(jax.lax.dot_general(
                        w, vh, (((1,), (0,)), ((), ())),
                        preferred_element_type=jnp.float32))
                ctx = jnp.concatenate(ctx_parts, axis=1)
                contrib = jax.lax.dot_general(
                    ctx, wo_c, (((1,), (0,)), ((), ())),
                    preferred_element_type=jnp.float32)
                out_ref[b] = out_ref[b] + contrib

        for h in range(N_DEV):
            j = lax.rem(my_i - h + N_DEV, N_DEV)
            if h > 0:
                r_wq, r_wo = make_pair(j, h - 1)
                r_wq.wait_recv()
                r_wo.wait_recv()
            if h < N_DEV - 1:
                s_wq, s_wo = make_pair(j, h)
                s_wq.start()
                s_wo.start()
            compute_chunk(j)
            if h < N_DEV - 1:
                s_wq.wait_send()
                s_wo.wait_send()

    return pl.pallas_call(
        body,
        out_shape=jax.ShapeDtypeStruct((B_LOC, SQ, D_MODEL), jnp.float32),
        in_specs=[pl.BlockSpec(memory_space=pltpu.VMEM)] * 5,
        out_specs=pl.BlockSpec(memory_space=pltpu.VMEM),
        scratch_shapes=[
            pltpu.VMEM((N_DEV, D_MODEL, CHUNK), jnp.float32),
            pltpu.VMEM((N_DEV, CHUNK, D_MODEL), jnp.float32),
            pltpu.SemaphoreType.DMA((N_DEV - 1,)),
            pltpu.SemaphoreType.DMA((N_DEV - 1,)),
            pltpu.SemaphoreType.DMA((N_DEV - 1,)),
            pltpu.SemaphoreType.DMA((N_DEV - 1,)),
        ],
        compiler_params=pltpu.CompilerParams(
            collective_id=0, vmem_limit_bytes=64 * 1024 * 1024),
    )(x, Wq, K_l, V_l, Wo)


# device time: 150547 ns/iter; 1.6034x vs baseline; 1.6034x over previous
import jax
import jax.numpy as jnp
from jax import lax
from jax.experimental import pallas as pl
from jax.experimental.pallas import tpu as pltpu

N_DEV = 16
B_LOC = 2
SQ = 256
SKV = 256
HQ = 64
DH = 64
D_MODEL = 512
H_PER = HQ // N_DEV
CHUNK = H_PER * DH
N_FWD = N_DEV // 2
N_BWD = N_DEV - 1 - N_FWD


def kernel(x, Wq, K_ext, V_ext, Wo):
    my = lax.axis_index("i")
    b0 = my * B_LOC
    K_l = jnp.transpose(lax.dynamic_slice_in_dim(K_ext, b0, B_LOC, 0), (0, 2, 1, 3))
    V_l = jnp.transpose(lax.dynamic_slice_in_dim(V_ext, b0, B_LOC, 0), (0, 2, 1, 3))

    def body(x_ref, wq_ref, k_ref, v_ref, wo_ref, out_ref,
             wq_g, wo_g, f_wq_s, f_wq_r, f_wo_s, f_wo_r,
             b_wq_s, b_wq_r, b_wo_s, b_wo_r):
        my_i = lax.axis_index("i")
        left = lax.rem(my_i + N_DEV - 1, N_DEV)
        right = lax.rem(my_i + 1, N_DEV)

        out_ref[...] = jnp.zeros((B_LOC, SQ, D_MODEL), jnp.float32)
        x2 = x_ref[...].reshape(B_LOC * SQ, D_MODEL)

        qi = lax.broadcasted_iota(jnp.int32, (SQ, SKV), 0)
        ki = lax.broadcasted_iota(jnp.int32, (SQ, SKV), 1)
        mask = (jnp.abs(qi - ki) <= 128) | (ki < 32) | (qi < 32)

        barrier = pltpu.get_barrier_semaphore()
        for nbr in (left, right):
            pl.semaphore_signal(barrier, inc=1, device_id=(nbr,),
                                device_id_type=pl.DeviceIdType.MESH)
        pl.semaphore_wait(barrier, 2)

        wq_g[my_i] = wq_ref[...]
        wo_g[my_i] = wo_ref[...]

        def make_pair(j, h, fwd):
            if fwd:
                tgt, wq_ss, wq_rs, wo_ss, wo_rs = right, f_wq_s, f_wq_r, f_wo_s, f_wo_r
            else:
                tgt, wq_ss, wq_rs, wo_ss, wo_rs = left, b_wq_s, b_wq_r, b_wo_s, b_wo_r
            r_wq = pltpu.make_async_remote_copy(
                src_ref=wq_g.at[j], dst_ref=wq_g.at[j],
                send_sem=wq_ss.at[h], recv_sem=wq_rs.at[h],
                device_id=(tgt,), device_id_type=pl.DeviceIdType.MESH)
            r_wo = pltpu.make_async_remote_copy(
                src_ref=wo_g.at[j], dst_ref=wo_g.at[j],
                send_sem=wo_ss.at[h], recv_sem=wo_rs.at[h],
                device_id=(tgt,), device_id_type=pl.DeviceIdType.MESH)
            return r_wq, r_wo

        def start_pair(j, h, fwd):
            r_wq, r_wo = make_pair(j, h, fwd)
            r_wq.start()
            r_wo.start()

        def wait_recv_pair(j, h, fwd):
            r_wq, r_wo = make_pair(j, h, fwd)
            r_wq.wait_recv()
            r_wo.wait_recv()

        def compute_chunk(j):
            wq_c = wq_g[j]
            wo_c = wo_g[j]
            qc2 = jax.lax.dot_general(
                x2, wq_c, (((1,), (0,)), ((), ())),
                preferred_element_type=jnp.float32)
            ctx_rows = []
            for b in range(B_LOC):
                qc = qc2[b * SQ:(b + 1) * SQ, :]
                ctx_parts = []
                for hh in range(H_PER):
                    hg = j * H_PER + hh
                    kh = k_ref[b, hg]
                    vh = v_ref[b, hg]
                    qh = qc[:, hh * DH:(hh + 1) * DH]
                    s = jax.lax.dot_general(
                        qh, kh, (((1,), (1,)), ((), ())),
                        preferred_element_type=jnp.float32) * 0.125
                    w = jnp.where(mask, jnp.exp(s), 0.0)
                    w = w / jnp.sum(w, axis=1, keepdims=True)
                    ctx_parts.append(jax.lax.dot_general(
                        w, vh, (((1,), (0,)), ((), ())),
                        preferred_element_type=jnp.float32))
                ctx_rows.append(jnp.concatenate(ctx_parts, axis=1))
            ctx2 = jnp.concatenate(ctx_rows, axis=0)
            contrib = jax.lax.dot_general(
                ctx2, wo_c, (((1,), (0,)), ((), ())),
                preferred_element_type=jnp.float32)
            out_ref[...] = out_ref[...] + contrib.reshape(B_LOC, SQ, D_MODEL)

        start_pair(my_i, 0, fwd=True)
        start_pair(my_i, 0, fwd=False)
        compute_chunk(my_i)

        for h in range(1, N_FWD + 1):
            jf = lax.rem(my_i - h + N_DEV, N_DEV)
            wait_recv_pair(jf, h - 1, fwd=True)
            if h < N_FWD:
                start_pair(jf, h, fwd=True)
            compute_chunk(jf)
            if h <= N_BWD:
                jb = lax.rem(my_i + h, N_DEV)
                wait_recv_pair(jb, h - 1, fwd=False)
                if h < N_BWD:
                    start_pair(jb, h, fwd=False)
                compute_chunk(jb)

        for h in range(N_FWD):
            jf = lax.rem(my_i - h + N_DEV, N_DEV)
            r_wq, r_wo = make_pair(jf, h, fwd=True)
            r_wq.wait_send()
            r_wo.wait_send()
        for h in range(N_BWD):
            jb = lax.rem(my_i + h, N_DEV)
            r_wq, r_wo = make_pair(jb, h, fwd=False)
            r_wq.wait_send()
            r_wo.wait_send()

    return pl.pallas_call(
        body,
        out_shape=jax.ShapeDtypeStruct((B_LOC, SQ, D_MODEL), jnp.float32),
        in_specs=[pl.BlockSpec(memory_space=pltpu.VMEM)] * 5,
        out_specs=pl.BlockSpec(memory_space=pltpu.VMEM),
        scratch_shapes=[
            pltpu.VMEM((N_DEV, D_MODEL, CHUNK), jnp.float32),
            pltpu.VMEM((N_DEV, CHUNK, D_MODEL), jnp.float32),
            pltpu.SemaphoreType.DMA((N_FWD,)),
            pltpu.SemaphoreType.DMA((N_FWD,)),
            pltpu.SemaphoreType.DMA((N_FWD,)),
            pltpu.SemaphoreType.DMA((N_FWD,)),
            pltpu.SemaphoreType.DMA((N_BWD,)),
            pltpu.SemaphoreType.DMA((N_BWD,)),
            pltpu.SemaphoreType.DMA((N_BWD,)),
            pltpu.SemaphoreType.DMA((N_BWD,)),
        ],
        compiler_params=pltpu.CompilerParams(
            collective_id=0, vmem_limit_bytes=64 * 1024 * 1024),
    )(x, Wq, K_l, V_l, Wo)


# device time: 96478 ns/iter; 2.5020x vs baseline; 1.5604x over previous
import jax
import jax.numpy as jnp
from jax import lax
from jax.experimental import pallas as pl
from jax.experimental.pallas import tpu as pltpu

N_DEV = 16
B_LOC = 2
SQ = 256
SKV = 256
HQ = 64
DH = 64
D_MODEL = 512
H_PER = HQ // N_DEV
CHUNK = H_PER * DH
N_FWD = N_DEV // 2
N_BWD = N_DEV - 1 - N_FWD

BF = jnp.bfloat16
F32 = jnp.float32


def kernel(x, Wq, K_ext, V_ext, Wo):
    my = lax.axis_index("i")
    b0 = my * B_LOC
    K_l = jnp.transpose(
        lax.dynamic_slice_in_dim(K_ext, b0, B_LOC, 0), (0, 2, 1, 3)).astype(BF)
    V_l = jnp.transpose(
        lax.dynamic_slice_in_dim(V_ext, b0, B_LOC, 0), (0, 2, 1, 3)).astype(BF)
    x_b = x.astype(BF)
    wq_b = Wq.astype(BF)
    wo_b = Wo.astype(BF)

    def body(x_ref, wq_ref, k_ref, v_ref, wo_ref, out_ref,
             wq_g, wo_g, f_wq_s, f_wq_r, f_wo_s, f_wo_r,
             b_wq_s, b_wq_r, b_wo_s, b_wo_r):
        my_i = lax.axis_index("i")
        left = lax.rem(my_i + N_DEV - 1, N_DEV)
        right = lax.rem(my_i + 1, N_DEV)

        out_ref[...] = jnp.zeros((B_LOC, SQ, D_MODEL), F32)
        x2 = x_ref[...].reshape(B_LOC * SQ, D_MODEL)

        qi = lax.broadcasted_iota(jnp.int32, (SQ, SKV), 0)
        ki = lax.broadcasted_iota(jnp.int32, (SQ, SKV), 1)
        mask = (jnp.abs(qi - ki) <= 128) | (ki < 32) | (qi < 32)

        barrier = pltpu.get_barrier_semaphore()
        for nbr in (left, right):
            pl.semaphore_signal(barrier, inc=1, device_id=(nbr,),
                                device_id_type=pl.DeviceIdType.MESH)
        pl.semaphore_wait(barrier, 2)

        wq_g[my_i] = wq_ref[...]
        wo_g[my_i] = wo_ref[...]

        def make_pair(j, h, fwd):
            if fwd:
                tgt, wq_ss, wq_rs, wo_ss, wo_rs = right, f_wq_s, f_wq_r, f_wo_s, f_wo_r
            else:
                tgt, wq_ss, wq_rs, wo_ss, wo_rs = left, b_wq_s, b_wq_r, b_wo_s, b_wo_r
            r_wq = pltpu.make_async_remote_copy(
                src_ref=wq_g.at[j], dst_ref=wq_g.at[j],
                send_sem=wq_ss.at[h], recv_sem=wq_rs.at[h],
                device_id=(tgt,), device_id_type=pl.DeviceIdType.MESH)
            r_wo = pltpu.make_async_remote_copy(
                src_ref=wo_g.at[j], dst_ref=wo_g.at[j],
                send_sem=wo_ss.at[h], recv_sem=wo_rs.at[h],
                device_id=(tgt,), device_id_type=pl.DeviceIdType.MESH)
            return r_wq, r_wo

        def start_pair(j, h, fwd):
            r_wq, r_wo = make_pair(j, h, fwd)
            r_wq.start()
            r_wo.start()

        def wait_recv_pair(j, h, fwd):
            r_wq, r_wo = make_pair(j, h, fwd)
            r_wq.wait_recv()
            r_wo.wait_recv()

        def compute_chunk(j):
            wq_c = wq_g[j]
            wo_c = wo_g[j]
            qc2 = jax.lax.dot_general(
                x2, wq_c, (((1,), (0,)), ((), ())),
                preferred_element_type=F32).astype(BF)
            ctx_rows = []
            for b in range(B_LOC):
                qc = qc2[b * SQ:(b + 1) * SQ, :]
                ctx_parts = []
                for hh in range(H_PER):
                    hg = j * H_PER + hh
                    kh = k_ref[b, hg]
                    vh = v_ref[b, hg]
                    qh = qc[:, hh * DH:(hh + 1) * DH]
                    s = jax.lax.dot_general(
                        qh, kh, (((1,), (1,)), ((), ())),
                        preferred_element_type=F32) * 0.125
                    w = jnp.where(mask, jnp.exp(s), 0.0)
                    recip = 1.0 / jnp.sum(w, axis=1, keepdims=True)
                    ctx_h = jax.lax.dot_general(
                        w.astype(BF), vh, (((1,), (0,)), ((), ())),
                        preferred_element_type=F32)
                    ctx_parts.append(ctx_h * recip)
                ctx_rows.append(jnp.concatenate(ctx_parts, axis=1))
            ctx2 = jnp.concatenate(ctx_rows, axis=0).astype(BF)
            contrib = jax.lax.dot_general(
                ctx2, wo_c, (((1,), (0,)), ((), ())),
                preferred_element_type=F32)
            out_ref[...] = out_ref[...] + contrib.reshape(B_LOC, SQ, D_MODEL)

        start_pair(my_i, 0, fwd=True)
        start_pair(my_i, 0, fwd=False)
        compute_chunk(my_i)

        for h in range(1, N_FWD + 1):
            jf = lax.rem(my_i - h + N_DEV, N_DEV)
            wait_recv_pair(jf, h - 1, fwd=True)
            if h < N_FWD:
                start_pair(jf, h, fwd=True)
            compute_chunk(jf)
            if h <= N_BWD:
                jb = lax.rem(my_i + h, N_DEV)
                wait_recv_pair(jb, h - 1, fwd=False)
                if h < N_BWD:
                    start_pair(jb, h, fwd=False)
                compute_chunk(jb)

        for h in range(N_FWD):
            jf = lax.rem(my_i - h + N_DEV, N_DEV)
            r_wq, r_wo = make_pair(jf, h, fwd=True)
            r_wq.wait_send()
            r_wo.wait_send()
        for h in range(N_BWD):
            jb = lax.rem(my_i + h, N_DEV)
            r_wq, r_wo = make_pair(jb, h, fwd=False)
            r_wq.wait_send()
            r_wo.wait_send()

    return pl.pallas_call(
        body,
        out_shape=jax.ShapeDtypeStruct((B_LOC, SQ, D_MODEL), F32),
        in_specs=[pl.BlockSpec(memory_space=pltpu.VMEM)] * 5,
        out_specs=pl.BlockSpec(memory_space=pltpu.VMEM),
        scratch_shapes=[
            pltpu.VMEM((N_DEV, D_MODEL, CHUNK), BF),
            pltpu.VMEM((N_DEV, CHUNK, D_MODEL), BF),
            pltpu.SemaphoreType.DMA((N_FWD,)),
            pltpu.SemaphoreType.DMA((N_FWD,)),
            pltpu.SemaphoreType.DMA((N_FWD,)),
            pltpu.SemaphoreType.DMA((N_FWD,)),
            pltpu.SemaphoreType.DMA((N_BWD,)),
            pltpu.SemaphoreType.DMA((N_BWD,)),
            pltpu.SemaphoreType.DMA((N_BWD,)),
            pltpu.SemaphoreType.DMA((N_BWD,)),
        ],
        compiler_params=pltpu.CompilerParams(
            collective_id=0, vmem_limit_bytes=64 * 1024 * 1024),
    )(x_b, wq_b, K_l, V_l, wo_b)


# device time: 93945 ns/iter; 2.5695x vs baseline; 1.0270x over previous
import jax
import jax.numpy as jnp
from jax import lax
from jax.experimental import pallas as pl
from jax.experimental.pallas import tpu as pltpu

N_DEV = 16
B_LOC = 2
SQ = 256
SKV = 256
HQ = 64
DH = 64
D_MODEL = 512
H_PER = HQ // N_DEV
CHUNK = H_PER * DH
N_FWD = N_DEV // 2
N_BWD = N_DEV - 1 - N_FWD

BF = jnp.bfloat16
F32 = jnp.float32

RING = [0, 4, 8, 12, 13, 9, 5, 1, 2, 6, 10, 14, 15, 11, 7, 3]
POS = [0] * N_DEV
for _r, _l in enumerate(RING):
    POS[_l] = _r
NEXT = [RING[(POS[_l] + 1) % N_DEV] for _l in range(N_DEV)]
PREV = [RING[(POS[_l] - 1) % N_DEV] for _l in range(N_DEV)]


def kernel(x, Wq, K_ext, V_ext, Wo):
    my = lax.axis_index("i")
    b0 = my * B_LOC
    K_l = jnp.transpose(
        lax.dynamic_slice_in_dim(K_ext, b0, B_LOC, 0), (0, 2, 1, 3)).astype(BF)
    V_l = jnp.transpose(
        lax.dynamic_slice_in_dim(V_ext, b0, B_LOC, 0), (0, 2, 1, 3)).astype(BF)
    V_l = jnp.concatenate(
        [V_l, jnp.ones((B_LOC, HQ, SKV, 1), BF)], axis=-1)
    x_b = x.astype(BF)
    wq_b = (Wq * 0.125).astype(BF)
    wo_b = Wo.astype(BF)
    nbr = jnp.stack([jnp.asarray(NEXT, jnp.int32)[my],
                     jnp.asarray(PREV, jnp.int32)[my],
                     jnp.asarray(POS, jnp.int32)[my]])
    ring_arr = jnp.asarray(RING, jnp.int32)

    def body(nbr_ref, ring_ref, x_ref, wq_ref, k_ref, v_ref, wo_ref, out_ref,
             wq_g, wo_g, f_wq_s, f_wq_r, f_wo_s, f_wo_r,
             b_wq_s, b_wq_r, b_wo_s, b_wo_r):
        my_i = lax.axis_index("i")
        right = nbr_ref[0]
        left = nbr_ref[1]
        my_pos = nbr_ref[2]

        out_ref[...] = jnp.zeros((B_LOC, SQ, D_MODEL), F32)
        x2 = x_ref[...].reshape(B_LOC * SQ, D_MODEL)

        qi = lax.broadcasted_iota(jnp.int32, (SQ, SKV), 0)
        ki = lax.broadcasted_iota(jnp.int32, (SQ, SKV), 1)
        mask = (jnp.abs(qi - ki) <= 128) | (ki < 32) | (qi < 32)

        barrier = pltpu.get_barrier_semaphore()
        for nbr in (left, right):
            pl.semaphore_signal(barrier, inc=1, device_id=(nbr,),
                                device_id_type=pl.DeviceIdType.MESH)
        pl.semaphore_wait(barrier, 2)

        wq_g[my_i] = wq_ref[...]
        wo_g[my_i] = wo_ref[...]

        def make_pieces(j, h, fwd):
            if fwd:
                tgt, wq_ss, wq_rs, wo_ss, wo_rs = right, f_wq_s, f_wq_r, f_wo_s, f_wo_r
            else:
                tgt, wq_ss, wq_rs, wo_ss, wo_rs = left, b_wq_s, b_wq_r, b_wo_s, b_wo_r
            descs = []
            for s in range(2):
                descs.append(pltpu.make_async_remote_copy(
                    src_ref=wq_g.at[j, pl.ds(s * 256, 256)],
                    dst_ref=wq_g.at[j, pl.ds(s * 256, 256)],
                    send_sem=wq_ss.at[h, s], recv_sem=wq_rs.at[h, s],
                    device_id=(tgt,), device_id_type=pl.DeviceIdType.MESH))
            for s in range(2):
                descs.append(pltpu.make_async_remote_copy(
                    src_ref=wo_g.at[j, pl.ds(s * 128, 128)],
                    dst_ref=wo_g.at[j, pl.ds(s * 128, 128)],
                    send_sem=wo_ss.at[h, s], recv_sem=wo_rs.at[h, s],
                    device_id=(tgt,), device_id_type=pl.DeviceIdType.MESH))
            return descs

        def start_pieces(j, h, fwd):
            for d in make_pieces(j, h, fwd):
                d.start()

        def recv_and_forward(j, h, fwd, forward):
            rdescs = make_pieces(j, h, fwd)
            sdescs = make_pieces(j, h + 1, fwd) if forward else [None] * 4
            for rd, sd in zip(rdescs, sdescs):
                rd.wait_recv()
                if sd is not None:
                    sd.start()

        def compute_chunk(j):
            wq_c = wq_g[j]
            wo_c = wo_g[j]
            qc2 = jax.lax.dot_general(
                x2, wq_c, (((1,), (0,)), ((), ())),
                preferred_element_type=F32).astype(BF)
            ctx_rows = []
            for b in range(B_LOC):
                qc = qc2[b * SQ:(b + 1) * SQ, :]
                ctx_parts = []
                for hh in range(H_PER):
                    hg = j * H_PER + hh
                    kh = k_ref[b, hg]
                    vh = v_ref[b, hg]
                    qh = qc[:, hh * DH:(hh + 1) * DH]
                    s = jax.lax.dot_general(
                        qh, kh, (((1,), (1,)), ((), ())),
                        preferred_element_type=F32)
                    w = jnp.where(mask, jnp.exp(s), 0.0)
                    ctx_aug = jax.lax.dot_general(
                        w.astype(BF), vh, (((1,), (0,)), ((), ())),
                        preferred_element_type=F32)
                    recip = 1.0 / ctx_aug[:, DH:DH + 1]
                    ctx_parts.append(ctx_aug[:, :DH] * recip)
                ctx_rows.append(jnp.concatenate(ctx_parts, axis=1))
            ctx2 = jnp.concatenate(ctx_rows, axis=0).astype(BF)
            contrib = jax.lax.dot_general(
                ctx2, wo_c, (((1,), (0,)), ((), ())),
                preferred_element_type=F32)
            out_ref[...] = out_ref[...] + contrib.reshape(B_LOC, SQ, D_MODEL)

        start_pieces(my_i, 0, fwd=True)
        start_pieces(my_i, 0, fwd=False)
        compute_chunk(my_i)

        for h in range(1, N_FWD + 1):
            jf = ring_ref[lax.rem(my_pos - h + N_DEV, N_DEV)]
            recv_and_forward(jf, h - 1, fwd=True, forward=h < N_FWD)
            jb = ring_ref[lax.rem(my_pos + h, N_DEV)]
            if h <= N_BWD:
                recv_and_forward(jb, h - 1, fwd=False, forward=h < N_BWD)
            compute_chunk(jf)
            if h <= N_BWD:
                compute_chunk(jb)

        for h in range(N_FWD):
            jf = ring_ref[lax.rem(my_pos - h + N_DEV, N_DEV)]
            for d in make_pieces(jf, h, fwd=True):
                d.wait_send()
        for h in range(N_BWD):
            jb = ring_ref[lax.rem(my_pos + h, N_DEV)]
            for d in make_pieces(jb, h, fwd=False):
                d.wait_send()

    return pl.pallas_call(
        body,
        out_shape=jax.ShapeDtypeStruct((B_LOC, SQ, D_MODEL), F32),
        in_specs=[pl.BlockSpec(memory_space=pltpu.SMEM)] * 2
        + [pl.BlockSpec(memory_space=pltpu.VMEM)] * 5,
        out_specs=pl.BlockSpec(memory_space=pltpu.VMEM),
        scratch_shapes=[
            pltpu.VMEM((N_DEV, D_MODEL, CHUNK), BF),
            pltpu.VMEM((N_DEV, CHUNK, D_MODEL), BF),
            pltpu.SemaphoreType.DMA((N_FWD, 2)),
            pltpu.SemaphoreType.DMA((N_FWD, 2)),
            pltpu.SemaphoreType.DMA((N_FWD, 2)),
            pltpu.SemaphoreType.DMA((N_FWD, 2)),
            pltpu.SemaphoreType.DMA((N_BWD, 2)),
            pltpu.SemaphoreType.DMA((N_BWD, 2)),
            pltpu.SemaphoreType.DMA((N_BWD, 2)),
            pltpu.SemaphoreType.DMA((N_BWD, 2)),
        ],
        compiler_params=pltpu.CompilerParams(
            collective_id=0, vmem_limit_bytes=64 * 1024 * 1024),
    )(nbr, ring_arr, x_b, wq_b, K_l, V_l, wo_b)
